# baseline (speedup 1.0000x reference)
import functools

import jax
import jax.numpy as jnp
from jax import lax
from jax.experimental import pallas as pl
from jax.experimental.pallas import tpu as pltpu
from jax.experimental.pallas import tpu_sc as plsc

N = 10000
D = 128
B = 512
L = 50
TOPK = 10
TEMP = 0.1
W_K = 10.0
CLW = 100.0

NB = 10112
DUMP_BIN = 10016
NR = 10112
DUMP_ROW = 10008
NT = 32
E0 = N * 32 + N
NCH = 82
EP = NT * NCH * 128
NCHG = 8
GP = NT * NCHG * 128

def _mesh():
    return plsc.VectorSubcoreMesh(core_axis_name="c", subcore_axis_name="s")


def _sc_hist(cols3d, z8):
    @functools.partial(
        pl.kernel,
        out_type=jax.ShapeDtypeStruct((NT, 8 * NB), jnp.float32),
        mesh=_mesh(),
        compiler_params=pltpu.CompilerParams(needs_layout_passes=False),
        scratch_types=[
            pltpu.VMEM((128,), jnp.int32),
            pltpu.VMEM((128,), jnp.int32),
            pltpu.VMEM((8 * NB,), jnp.float32),
            pltpu.SemaphoreType.DMA,
            pltpu.SemaphoreType.DMA,
        ],
    )
    def k(cols_hbm, z_hbm, out_hbm, idx_v0, idx_v1, hist_v, is0, is1):
        wid = lax.axis_index("s") * 2 + lax.axis_index("c")
        idx_v = (idx_v0, idx_v1)
        isem = (is0, is1)
        pltpu.sync_copy(z_hbm, hist_v)
        lane = lax.iota(jnp.int32, 16)
        mask_lo = lane < 8
        mask_hi = jnp.logical_not(mask_lo)
        off_lo = lane * NB
        off_hi = (lane - 8) * NB
        ones = jnp.ones((16,), jnp.float32)
        base0 = wid * NCH
        pltpu.async_copy(cols_hbm.at[base0], idx_v0, is0)

        def pair(j0, carry):
            for b in range(2):
                nb = 1 - b
                ch = j0 * 2 + b

                @pl.when(ch + 1 < NCH)
                def _():
                    pltpu.async_copy(cols_hbm.at[base0 + ch + 1],
                                     idx_v[nb], isem[nb])

                pltpu.make_async_copy(cols_hbm.at[base0], idx_v[b],
                                      isem[b]).wait()
                for j in range(8):
                    idx16 = idx_v[b][pl.ds(j * 16, 16)]
                    plsc.addupdate_scatter(hist_v, [off_lo + idx16], ones,
                                           mask=mask_lo)
                    plsc.addupdate_scatter(hist_v, [off_hi + idx16], ones,
                                           mask=mask_hi)
            return carry

        lax.fori_loop(0, NCH // 2, pair, 0)
        pltpu.sync_copy(hist_v, out_hbm.at[wid])

    return k(cols3d, z8)


def _sc_spmm(g, cols2d, rows2d, z632):
    @functools.partial(
        pl.kernel,
        out_type=jax.ShapeDtypeStruct((2, NR, D), jnp.float32),
        mesh=_mesh(),
        compiler_params=pltpu.CompilerParams(needs_layout_passes=False),
        scratch_types=[
            pltpu.VMEM((128,), jnp.int32),
            pltpu.VMEM((128,), jnp.int32),
            pltpu.VMEM((128,), jnp.int32),
            pltpu.VMEM((128,), jnp.int32),
            pltpu.VMEM((128, D), jnp.float32),
            pltpu.VMEM((128, D), jnp.float32),
            pltpu.VMEM_SHARED((NR, D), jnp.float32),
            pltpu.SemaphoreType.DMA,
            pltpu.SemaphoreType.DMA,
            pltpu.SemaphoreType.DMA,
            pltpu.SemaphoreType.DMA,
        ],
    )
    def k(g_hbm, c_hbm, r_hbm, z_hbm, out_hbm, colv0, colv1, rowv0, rowv1,
          buf0, buf1, acc, gs0, gs1, ss0, ss1):
        cid = lax.axis_index("c")
        sid = lax.axis_index("s")
        wid = sid * 2 + cid
        colv = (colv0, colv1)
        rowv = (rowv0, rowv1)
        buf = (buf0, buf1)
        gsem = (gs0, gs1)
        ssem = (ss0, ss1)
        pltpu.sync_copy(z_hbm, acc.at[pl.ds(sid * 632, 632)])
        plsc.subcore_barrier()
        base0 = wid * NCH
        pltpu.sync_copy(c_hbm.at[base0], colv0)
        pltpu.sync_copy(r_hbm.at[base0], rowv0)
        pltpu.async_copy(g_hbm.at[colv0], buf0, gs0)

        def pair(j0, carry):
            for b in range(2):
                nb = 1 - b
                j = j0 * 2 + b

                @pl.when(j + 1 < NCH)
                def _():
                    @pl.when(j >= 1)
                    def _():
                        pltpu.make_async_copy(buf[nb], acc.at[rowv[nb]],
                                              ssem[nb]).wait()

                    pltpu.sync_copy(c_hbm.at[base0 + j + 1], colv[nb])
                    pltpu.sync_copy(r_hbm.at[base0 + j + 1], rowv[nb])
                    pltpu.async_copy(g_hbm.at[colv[nb]], buf[nb], gsem[nb])

                pltpu.make_async_copy(g_hbm.at[colv[b]], buf[b],
                                      gsem[b]).wait()
                pltpu.async_copy(buf[b], acc.at[rowv[b]], ssem[b], add=True)
            return carry

        lax.fori_loop(0, NCH // 2, pair, 0)
        pltpu.make_async_copy(buf0, acc.at[rowv0], ss0).wait()
        pltpu.make_async_copy(buf1, acc.at[rowv1], ss1).wait()
        plsc.subcore_barrier()
        pltpu.sync_copy(acc.at[pl.ds(sid * 632, 632)],
                        out_hbm.at[cid, pl.ds(sid * 632, 632)])

    return k(g, cols2d, rows2d, z632)


def _sc_gather(tbl, gidx2d):
    @functools.partial(
        pl.kernel,
        out_type=jax.ShapeDtypeStruct((NT * NCHG, 128, D), jnp.float32),
        mesh=_mesh(),
        compiler_params=pltpu.CompilerParams(needs_layout_passes=False),
        scratch_types=[
            pltpu.VMEM((128,), jnp.int32),
            pltpu.VMEM((128,), jnp.int32),
            pltpu.VMEM((128, D), jnp.float32),
            pltpu.VMEM((128, D), jnp.float32),
            pltpu.SemaphoreType.DMA,
            pltpu.SemaphoreType.DMA,
            pltpu.SemaphoreType.DMA,
            pltpu.SemaphoreType.DMA,
        ],
    )
    def k(tbl_hbm, gidx_hbm, out_hbm, idxv0, idxv1, buf0, buf1,
          gs0, gs1, ws0, ws1):
        wid = lax.axis_index("s") * 2 + lax.axis_index("c")
        idxv = (idxv0, idxv1)
        buf = (buf0, buf1)
        gsem = (gs0, gs1)
        wsem = (ws0, ws1)
        base0 = wid * NCHG
        pltpu.sync_copy(gidx_hbm.at[base0], idxv0)
        pltpu.async_copy(tbl_hbm.at[idxv0], buf0, gs0)

        def pair(j0, carry):
            for b in range(2):
                nb = 1 - b
                j = j0 * 2 + b

                @pl.when(j + 1 < NCHG)
                def _():
                    @pl.when(j >= 1)
                    def _():
                        pltpu.make_async_copy(buf[nb],
                                              out_hbm.at[base0 + j - 1],
                                              wsem[nb]).wait()

                    pltpu.sync_copy(gidx_hbm.at[base0 + j + 1], idxv[nb])
                    pltpu.async_copy(tbl_hbm.at[idxv[nb]], buf[nb], gsem[nb])

                pltpu.make_async_copy(tbl_hbm.at[idxv[b]], buf[b],
                                      gsem[b]).wait()
                pltpu.async_copy(buf[b], out_hbm.at[base0 + j], wsem[b])
            return carry

        lax.fori_loop(0, NCHG // 2, pair, 0)
        pltpu.make_async_copy(buf0, out_hbm.at[base0], ws0).wait()
        pltpu.make_async_copy(buf1, out_hbm.at[base0], ws1).wait()

    return k(tbl, gidx2d)


def _l2n(x):
    n = jnp.sqrt(jnp.sum(x * x, axis=-1, keepdims=True))
    return x / jnp.maximum(n, 1e-12)


def _attw(emb, att_w, att_b):
    s = lax.dot_general(emb, att_w, (((1,), (1,)), ((), ())))[:, 0:1] + att_b
    m = jnp.max(s, axis=0, keepdims=True)
    e = jnp.exp(s - m)
    return e / jnp.sum(e, axis=0, keepdims=True)


def _prep_g(emb, w, att_w, att_b, inv_deg):
    h = lax.dot_general(emb, w, (((1,), (1,)), ((), ())))
    return h * _attw(emb, att_w, att_b) * inv_deg


def _k1_body(hist_ref, emb_ref, w_ref, aw_ref, ab_ref, g_ref, invdeg_ref):
    ones = jnp.ones((NT * 8, 8), jnp.float32)
    deg = lax.dot_general(hist_ref[...], ones, (((0,), (0,)), ((), ())))
    inv_deg = 1.0 / deg[:N, 0:1]
    invdeg_ref[...] = inv_deg
    g_ref[...] = _prep_g(emb_ref[...], w_ref[...], aw_ref[...],
                         ab_ref[0, 0], inv_deg)


def _tc_prep0(hist, emb, w0, att_w, att_b):
    return pl.pallas_call(
        _k1_body,
        out_shape=[jax.ShapeDtypeStruct((N, D), jnp.float32),
                   jax.ShapeDtypeStruct((N, 1), jnp.float32)],
    )(hist, emb, w0, att_w, att_b)


def _k2_body(p_ref, w_ref, aw_ref, ab_ref, invdeg_ref, hn_ref, g_ref):
    h = p_ref[0, :N, :] + p_ref[1, :N, :]
    hn = _l2n(h)
    hn_ref[...] = hn
    g_ref[...] = _prep_g(hn, w_ref[...], aw_ref[...], ab_ref[0, 0],
                         invdeg_ref[...])


def _tc_prep1(p, w1, att_w, att_b, inv_deg):
    return pl.pallas_call(
        _k2_body,
        out_shape=[jax.ShapeDtypeStruct((N, D), jnp.float32),
                   jax.ShapeDtypeStruct((N, D), jnp.float32)],
    )(p, w1, att_w, att_b, inv_deg)


def _k3_body(p_ref, emb_ref, hn1_ref, embi_ref, itemsn_ref):
    h2 = p_ref[0, :N, :] + p_ref[1, :N, :]
    hn2 = _l2n(h2)
    emb_i = (emb_ref[...] + hn1_ref[...] + hn2) * (1.0 / 3.0)
    embi_ref[...] = emb_i
    itemsn_ref[...] = _l2n(emb_i)


def _tc_final_items(p, emb, hn1):
    return pl.pallas_call(
        _k3_body,
        out_shape=[jax.ShapeDtypeStruct((N, D), jnp.float32),
                   jax.ShapeDtypeStruct((N, D), jnp.float32)],
    )(p, emb, hn1)


SB = 128
SR = SB * L


def _k4_body(sh_ref, vm_ref, mk_ref, len_ref, pos_ref, w1a_ref, w1b_ref,
             g1w_ref, g1b_ref, g2w_ref, w2_ref, sess_ref):
    sh = sh_ref[...] * vm_ref[...]
    sh3 = jnp.reshape(sh, (SB, L, D))
    hs = jnp.sum(sh3, axis=1) / len_ref[...]
    pos_part = lax.dot_general(pos_ref[:L, :], w1a_ref[...],
                               (((1,), (0,)), ((), ())))
    nh_lin = lax.dot_general(sh, w1b_ref[...], (((1,), (0,)), ((), ())))
    nh_lin = nh_lin + jnp.reshape(
        jnp.broadcast_to(pos_part[None], (SB, L, D)), (SR, D))
    nh = jnp.tanh(nh_lin)
    hsg = lax.dot_general(hs, g2w_ref[...], (((1,), (1,)), ((), ())))
    hsg = jnp.reshape(jnp.broadcast_to(hsg[:, None, :], (SB, L, D)), (SR, D))
    pre = lax.dot_general(nh, g1w_ref[...], (((1,), (1,)), ((), ())))
    nh2 = jax.nn.sigmoid(pre + g1b_ref[...] + hsg)
    beta = lax.dot_general(nh2, w2_ref[...], (((1,), (0,)), ((), ())))[:, 0:1]
    beta = beta * mk_ref[...]
    sel = jnp.sum(jnp.reshape(beta * sh, (SB, L, D)), axis=1)
    sess_ref[...] = W_K * _l2n(sel)


def _tc_session(seq_h, vmask, maskf, slen, pos, w1a, w1b, g1w, g1b, g2w, w2):
    nblk = B // SB
    full = lambda s: pl.BlockSpec(s, lambda i: (0, 0))
    return pl.pallas_call(
        _k4_body,
        grid=(nblk,),
        in_specs=[
            pl.BlockSpec((SR, D), lambda i: (i, 0)),
            pl.BlockSpec((SR, 1), lambda i: (i, 0)),
            pl.BlockSpec((SR, 1), lambda i: (i, 0)),
            pl.BlockSpec((SB, 1), lambda i: (i, 0)),
            full((200, D)), full((D, D)), full((D, D)), full((D, D)),
            full((1, D)), full((D, D)), full((D, 8)),
        ],
        out_specs=pl.BlockSpec((SB, D), lambda i: (i, 0)),
        out_shape=jax.ShapeDtypeStruct((B, D), jnp.float32),
    )(seq_h, vmask, maskf, slen, pos, w1a, w1b, g1w, g1b, g2w, w2)


def _k5_body(sess_ref, items_ref, tar_ref, scores_ref, loss_ref):
    sc = lax.dot_general(sess_ref[...], items_ref[...],
                         (((1,), (1,)), ((), ())))
    scores_ref[...] = sc
    m = jnp.max(sc, axis=1, keepdims=True)
    lse = jnp.log(jnp.sum(jnp.exp(sc - m), axis=1, keepdims=True)) + m
    col = lax.broadcasted_iota(jnp.int32, (B, N), 1)
    pick = jnp.sum(jnp.where(col == tar_ref[...], sc, 0.0), axis=1,
                   keepdims=True)
    loss_ref[...] = jnp.reshape(jnp.mean(lse - pick), (1, 1))


def _tc_scores(sess, items_n, tar2d):
    return pl.pallas_call(
        _k5_body,
        out_shape=[jax.ShapeDtypeStruct((B, N), jnp.float32),
                   jax.ShapeDtypeStruct((1, 1), jnp.float32)],
    )(sess, items_n, tar2d)


BR = 400


def _k6_body(x_ref, items_ref, cl_ref):
    lg = lax.dot_general(x_ref[...], items_ref[...],
                         (((1,), (1,)), ((), ()))) * (1.0 / TEMP)
    m0 = jnp.max(lg, axis=1, keepdims=True)
    acc = jnp.ones((BR, 1), jnp.float32)
    cur = lg
    mprev = m0
    for _ in range(TOPK - 1):
        cur = jnp.where(cur == mprev, -1e30, cur)
        mk = jnp.max(cur, axis=1, keepdims=True)
        acc = acc + jnp.exp(mk - m0)
        mprev = mk
    part = jnp.reshape(jnp.sum(jnp.log(acc)) * (CLW / N), (1, 1))

    @pl.when(pl.program_id(0) == 0)
    def _():
        cl_ref[...] = jnp.zeros((1, 1), jnp.float32)

    cl_ref[...] += part


def _tc_cl(items_n):
    return pl.pallas_call(
        _k6_body,
        grid=(N // BR,),
        in_specs=[
            pl.BlockSpec((BR, D), lambda i: (i, 0)),
            pl.BlockSpec((N, D), lambda i: (0, 0)),
        ],
        out_specs=pl.BlockSpec((1, 1), lambda i: (0, 0)),
        out_shape=jax.ShapeDtypeStruct((1, 1), jnp.float32),
    )(items_n, items_n)


def _pad2(x, r, c):
    return jnp.pad(x, ((0, r - x.shape[0]), (0, c - x.shape[1])))


def kernel(embedding, pos_embedding, w_1, w_2, glu1_W, glu1_b, glu2_W,
           att_W, att_b, w_item0, w_item1, adj_vals, adj_row, adj_col,
           session_item, session_len, reversed_sess_item, mask, tar):
    f32 = jnp.float32
    emb = _pad2(embedding, N, D)
    pos = _pad2(pos_embedding, 200, D)
    w1a = _pad2(w_1[:100, :], D, D)
    w1b = _pad2(w_1[100:, :], D, D)
    w2 = _pad2(w_2, D, 8)
    g1w = _pad2(glu1_W, D, D)
    g1b = _pad2(glu1_b[None, :], 1, D)
    g2w = _pad2(glu2_W, D, D)
    aw = _pad2(att_W, 8, D)
    ab = att_b.reshape(1, 1)
    w0 = _pad2(w_item0, D, D)
    w1 = _pad2(w_item1, D, D)

    hist_cols = jnp.concatenate(
        [adj_col, jnp.full((EP - E0,), DUMP_BIN, jnp.int32)]
    ).reshape(NT * NCH, 128)
    z8 = jnp.zeros((8 * NB,), f32)
    hist = _sc_hist(hist_cols, z8).reshape(NT * 8, NB)

    cols2d = jnp.concatenate(
        [adj_col, jnp.zeros((EP - E0,), jnp.int32)]).reshape(NT * NCH, 128)
    rows2d = jnp.concatenate(
        [adj_row, jnp.full((EP - E0,), DUMP_ROW, jnp.int32)]
    ).reshape(NT * NCH, 128)
    z632 = jnp.zeros((632, D), f32)

    g0, inv_deg = _tc_prep0(hist, emb, w0, aw, ab)
    p0 = _sc_spmm(g0, cols2d, rows2d, z632)
    hn1, g1 = _tc_prep1(p0, w1, aw, ab, inv_deg)
    p1 = _sc_spmm(g1, cols2d, rows2d, z632)
    emb_i, items_n = _tc_final_items(p1, emb, hn1)

    rsi = reversed_sess_item.reshape(B * L)
    gidx = jnp.maximum(rsi - 1, 0)
    gidx2d = jnp.concatenate(
        [gidx, jnp.zeros((GP - B * L,), jnp.int32)]).reshape(NT * NCHG, 128)
    seq_h = _sc_gather(emb_i, gidx2d).reshape(GP, D)[:B * L]

    vmask = (rsi > 0).astype(f32).reshape(B * L, 1)
    maskf = mask.reshape(B * L, 1)
    sess = _tc_session(seq_h, vmask, maskf, session_len, pos, w1a, w1b,
                       g1w, g1b, g2w, w2)

    scores, loss_item = _tc_scores(sess, items_n, tar.reshape(B, 1))
    cl = _tc_cl(items_n)
    return (loss_item[0, 0], scores, cl[0, 0])


# speedup vs baseline: 1.1838x; 1.1838x over previous
import functools

import jax
import jax.numpy as jnp
from jax import lax
from jax.experimental import pallas as pl
from jax.experimental.pallas import tpu as pltpu
from jax.experimental.pallas import tpu_sc as plsc

N = 10000
D = 128
B = 512
L = 50
TOPK = 10
TEMP = 0.1
W_K = 10.0
CLW = 100.0

NB = 10112
DUMP_BIN = 10016
NR = 10112
DUMP_ROW = 10008
NT = 32
E0 = N * 32 + N
NCH = 82
EP = NT * NCH * 128
NCHG = 8
GP = NT * NCHG * 128

def _mesh():
    return plsc.VectorSubcoreMesh(core_axis_name="c", subcore_axis_name="s")


def _sc_hist(cols3d, z8):
    @functools.partial(
        pl.kernel,
        out_type=jax.ShapeDtypeStruct((NT, 8 * NB), jnp.float32),
        mesh=_mesh(),
        compiler_params=pltpu.CompilerParams(needs_layout_passes=False),
        scratch_types=[
            pltpu.VMEM((NCH, 128), jnp.int32),
            pltpu.VMEM((8 * NB,), jnp.float32),
        ],
    )
    def k(cols_hbm, z_hbm, out_hbm, iall_v, hist_v):
        wid = lax.axis_index("s") * 2 + lax.axis_index("c")
        pltpu.sync_copy(cols_hbm.at[wid], iall_v)
        pltpu.sync_copy(z_hbm, hist_v)
        lane = lax.iota(jnp.int32, 16)
        mask_lo = lane < 8
        mask_hi = jnp.logical_not(mask_lo)
        off_lo = lane * NB
        off_hi = (lane - 8) * NB
        ones = jnp.ones((16,), jnp.float32)

        def chunk(ch, carry):
            for j in range(8):
                idx16 = iall_v[ch, pl.ds(j * 16, 16)]
                plsc.addupdate_scatter(hist_v, [off_lo + idx16], ones,
                                       mask=mask_lo)
                plsc.addupdate_scatter(hist_v, [off_hi + idx16], ones,
                                       mask=mask_hi)
            return carry

        lax.fori_loop(0, NCH, chunk, 0)
        pltpu.sync_copy(hist_v, out_hbm.at[wid])

    return k(cols3d, z8)


def _sc_spmm(g, packed, z632):
    @functools.partial(
        pl.kernel,
        out_type=jax.ShapeDtypeStruct((2, NR, D), jnp.float32),
        mesh=_mesh(),
        compiler_params=pltpu.CompilerParams(needs_layout_passes=False),
        scratch_types=[
            pltpu.VMEM((NCH, 128), jnp.int32),
            pltpu.VMEM((128,), jnp.int32),
            pltpu.VMEM((128,), jnp.int32),
            pltpu.VMEM((128,), jnp.int32),
            pltpu.VMEM((128,), jnp.int32),
            pltpu.VMEM((128, D), jnp.float32),
            pltpu.VMEM((128, D), jnp.float32),
            pltpu.VMEM_SHARED((NR, D), jnp.float32),
            pltpu.SemaphoreType.DMA,
            pltpu.SemaphoreType.DMA,
            pltpu.SemaphoreType.DMA,
            pltpu.SemaphoreType.DMA,
        ],
    )
    def k(g_hbm, p_hbm, z_hbm, out_hbm, pall_v, colv0, colv1, rowv0, rowv1,
          buf0, buf1, acc, gs0, gs1, ss0, ss1):
        cid = lax.axis_index("c")
        sid = lax.axis_index("s")
        wid = sid * 2 + cid
        colv = (colv0, colv1)
        rowv = (rowv0, rowv1)
        buf = (buf0, buf1)
        gsem = (gs0, gs1)
        ssem = (ss0, ss1)
        pltpu.sync_copy(p_hbm.at[wid], pall_v)
        pltpu.sync_copy(z_hbm, acc.at[pl.ds(sid * 632, 632)])

        def unpack(j, b):
            for t in range(8):
                pk = pall_v[j, pl.ds(t * 16, 16)]
                colv[b][pl.ds(t * 16, 16)] = jnp.bitwise_and(pk, 16383)
                rowv[b][pl.ds(t * 16, 16)] = jnp.right_shift(pk, 14)

        plsc.subcore_barrier()
        unpack(0, 0)
        pltpu.async_copy(g_hbm.at[colv0], buf0, gs0)

        def pair(j0, carry):
            for b in range(2):
                nb = 1 - b
                j = j0 * 2 + b

                @pl.when(j + 1 < NCH)
                def _():
                    @pl.when(j >= 1)
                    def _():
                        pltpu.make_async_copy(buf[nb], acc.at[rowv[nb]],
                                              ssem[nb]).wait()

                    unpack(j + 1, nb)
                    pltpu.async_copy(g_hbm.at[colv[nb]], buf[nb], gsem[nb])

                pltpu.make_async_copy(g_hbm.at[colv[b]], buf[b],
                                      gsem[b]).wait()
                pltpu.async_copy(buf[b], acc.at[rowv[b]], ssem[b],
                                add=True)
            return carry

        lax.fori_loop(0, NCH // 2, pair, 0)
        pltpu.make_async_copy(buf0, acc.at[rowv0], ss0).wait()
        pltpu.make_async_copy(buf1, acc.at[rowv1], ss1).wait()
        plsc.subcore_barrier()
        pltpu.sync_copy(acc.at[pl.ds(sid * 632, 632)],
                        out_hbm.at[cid, pl.ds(sid * 632, 632)])

    return k(g, packed, z632)


def _sc_gather(tbl, gidx2d):
    @functools.partial(
        pl.kernel,
        out_type=jax.ShapeDtypeStruct((NT * NCHG, 128, D), jnp.float32),
        mesh=_mesh(),
        compiler_params=pltpu.CompilerParams(needs_layout_passes=False),
        scratch_types=[
            pltpu.VMEM((NCHG, 128), jnp.int32),
            pltpu.VMEM((128, D), jnp.float32),
            pltpu.VMEM((128, D), jnp.float32),
            pltpu.SemaphoreType.DMA,
            pltpu.SemaphoreType.DMA,
            pltpu.SemaphoreType.DMA,
            pltpu.SemaphoreType.DMA,
        ],
    )
    def k(tbl_hbm, gidx_hbm, out_hbm, iall_v, buf0, buf1,
          gs0, gs1, ws0, ws1):
        wid = lax.axis_index("s") * 2 + lax.axis_index("c")
        buf = (buf0, buf1)
        gsem = (gs0, gs1)
        wsem = (ws0, ws1)
        base0 = wid * NCHG
        pltpu.sync_copy(gidx_hbm.at[wid], iall_v)
        pltpu.async_copy(tbl_hbm.at[iall_v.at[0]], buf0, gs0)

        def pair(j0, carry):
            for b in range(2):
                nb = 1 - b
                j = j0 * 2 + b

                @pl.when(j + 1 < NCHG)
                def _():
                    @pl.when(j >= 1)
                    def _():
                        pltpu.make_async_copy(buf[nb],
                                              out_hbm.at[base0],
                                              wsem[nb]).wait()

                    pltpu.async_copy(tbl_hbm.at[iall_v.at[j + 1]],
                                     buf[nb], gsem[nb])

                pltpu.make_async_copy(tbl_hbm.at[iall_v.at[j]], buf[b],
                                      gsem[b]).wait()
                pltpu.async_copy(buf[b], out_hbm.at[base0 + j], wsem[b])
            return carry

        lax.fori_loop(0, NCHG // 2, pair, 0)
        pltpu.make_async_copy(buf0, out_hbm.at[base0], ws0).wait()
        pltpu.make_async_copy(buf1, out_hbm.at[base0], ws1).wait()

    return k(tbl, gidx2d)


def _l2n(x):
    n = jnp.sqrt(jnp.sum(x * x, axis=-1, keepdims=True))
    return x / jnp.maximum(n, 1e-12)


def _attw(emb, att_w, att_b):
    s = lax.dot_general(emb, att_w, (((1,), (1,)), ((), ())))[:, 0:1] + att_b
    m = jnp.max(s, axis=0, keepdims=True)
    e = jnp.exp(s - m)
    return e / jnp.sum(e, axis=0, keepdims=True)


def _prep_g(emb, w, att_w, att_b, inv_deg):
    h = lax.dot_general(emb, w, (((1,), (1,)), ((), ())))
    return h * _attw(emb, att_w, att_b) * inv_deg


def _k1_body(hist_ref, emb_ref, w_ref, aw_ref, ab_ref, g_ref, invdeg_ref):
    ones = jnp.ones((NT * 8, 8), jnp.float32)
    deg = lax.dot_general(hist_ref[...], ones, (((0,), (0,)), ((), ())))
    inv_deg = 1.0 / deg[:N, 0:1]
    invdeg_ref[...] = inv_deg
    g_ref[...] = _prep_g(emb_ref[...], w_ref[...], aw_ref[...],
                         ab_ref[0, 0], inv_deg)


def _tc_prep0(hist, emb, w0, att_w, att_b):
    return pl.pallas_call(
        _k1_body,
        out_shape=[jax.ShapeDtypeStruct((N, D), jnp.float32),
                   jax.ShapeDtypeStruct((N, 1), jnp.float32)],
    )(hist, emb, w0, att_w, att_b)


def _k2_body(p_ref, w_ref, aw_ref, ab_ref, invdeg_ref, hn_ref, g_ref):
    h = p_ref[0, :N, :] + p_ref[1, :N, :]
    hn = _l2n(h)
    hn_ref[...] = hn
    g_ref[...] = _prep_g(hn, w_ref[...], aw_ref[...], ab_ref[0, 0],
                         invdeg_ref[...])


def _tc_prep1(p, w1, att_w, att_b, inv_deg):
    return pl.pallas_call(
        _k2_body,
        out_shape=[jax.ShapeDtypeStruct((N, D), jnp.float32),
                   jax.ShapeDtypeStruct((N, D), jnp.float32)],
    )(p, w1, att_w, att_b, inv_deg)


def _k3_body(p_ref, emb_ref, hn1_ref, embi_ref, itemsn_ref):
    h2 = p_ref[0, :N, :] + p_ref[1, :N, :]
    hn2 = _l2n(h2)
    emb_i = (emb_ref[...] + hn1_ref[...] + hn2) * (1.0 / 3.0)
    embi_ref[...] = emb_i
    itemsn_ref[...] = _l2n(emb_i)


def _tc_final_items(p, emb, hn1):
    return pl.pallas_call(
        _k3_body,
        out_shape=[jax.ShapeDtypeStruct((N, D), jnp.float32),
                   jax.ShapeDtypeStruct((N, D), jnp.float32)],
    )(p, emb, hn1)


SB = 128
SR = SB * L


def _k4_body(sh_ref, vm_ref, mk_ref, len_ref, pos_ref, w1a_ref, w1b_ref,
             g1w_ref, g1b_ref, g2w_ref, w2_ref, sess_ref):
    sh = sh_ref[...] * vm_ref[...]
    sh3 = jnp.reshape(sh, (SB, L, D))
    hs = jnp.sum(sh3, axis=1) / len_ref[...]
    pos_part = lax.dot_general(pos_ref[:L, :], w1a_ref[...],
                               (((1,), (0,)), ((), ())))
    nh_lin = lax.dot_general(sh, w1b_ref[...], (((1,), (0,)), ((), ())))
    nh_lin = nh_lin + jnp.reshape(
        jnp.broadcast_to(pos_part[None], (SB, L, D)), (SR, D))
    nh = jnp.tanh(nh_lin)
    hsg = lax.dot_general(hs, g2w_ref[...], (((1,), (1,)), ((), ())))
    hsg = jnp.reshape(jnp.broadcast_to(hsg[:, None, :], (SB, L, D)), (SR, D))
    pre = lax.dot_general(nh, g1w_ref[...], (((1,), (1,)), ((), ())))
    nh2 = jax.nn.sigmoid(pre + g1b_ref[...] + hsg)
    beta = lax.dot_general(nh2, w2_ref[...], (((1,), (0,)), ((), ())))[:, 0:1]
    beta = beta * mk_ref[...]
    sel = jnp.sum(jnp.reshape(beta * sh, (SB, L, D)), axis=1)
    sess_ref[...] = W_K * _l2n(sel)


def _tc_session(seq_h, vmask, maskf, slen, pos, w1a, w1b, g1w, g1b, g2w, w2):
    nblk = B // SB
    full = lambda s: pl.BlockSpec(s, lambda i: (0, 0))
    return pl.pallas_call(
        _k4_body,
        grid=(nblk,),
        in_specs=[
            pl.BlockSpec((SR, D), lambda i: (i, 0)),
            pl.BlockSpec((SR, 1), lambda i: (i, 0)),
            pl.BlockSpec((SR, 1), lambda i: (i, 0)),
            pl.BlockSpec((SB, 1), lambda i: (i, 0)),
            full((200, D)), full((D, D)), full((D, D)), full((D, D)),
            full((1, D)), full((D, D)), full((D, 8)),
        ],
        out_specs=pl.BlockSpec((SB, D), lambda i: (i, 0)),
        out_shape=jax.ShapeDtypeStruct((B, D), jnp.float32),
    )(seq_h, vmask, maskf, slen, pos, w1a, w1b, g1w, g1b, g2w, w2)


def _k5_body(sess_ref, items_ref, tar_ref, scores_ref, loss_ref):
    sc = lax.dot_general(sess_ref[...], items_ref[...],
                         (((1,), (1,)), ((), ())))
    scores_ref[...] = sc
    m = jnp.max(sc, axis=1, keepdims=True)
    lse = jnp.log(jnp.sum(jnp.exp(sc - m), axis=1, keepdims=True)) + m
    col = lax.broadcasted_iota(jnp.int32, (B, N), 1)
    pick = jnp.sum(jnp.where(col == tar_ref[...], sc, 0.0), axis=1,
                   keepdims=True)
    loss_ref[...] = jnp.reshape(jnp.mean(lse - pick), (1, 1))


def _tc_scores(sess, items_n, tar2d):
    return pl.pallas_call(
        _k5_body,
        out_shape=[jax.ShapeDtypeStruct((B, N), jnp.float32),
                   jax.ShapeDtypeStruct((1, 1), jnp.float32)],
    )(sess, items_n, tar2d)


BR = 400


def _k6_body(x_ref, items_ref, cl_ref):
    lg = lax.dot_general(x_ref[...], items_ref[...],
                         (((1,), (1,)), ((), ()))) * (1.0 / TEMP)
    m0 = jnp.max(lg, axis=1, keepdims=True)
    acc = jnp.ones((BR, 1), jnp.float32)
    cur = lg
    mprev = m0
    for _ in range(TOPK - 1):
        cur = jnp.where(cur == mprev, -1e30, cur)
        mk = jnp.max(cur, axis=1, keepdims=True)
        acc = acc + jnp.exp(mk - m0)
        mprev = mk
    part = jnp.reshape(jnp.sum(jnp.log(acc)) * (CLW / N), (1, 1))

    @pl.when(pl.program_id(0) == 0)
    def _():
        cl_ref[...] = jnp.zeros((1, 1), jnp.float32)

    cl_ref[...] += part


def _tc_cl(items_n):
    return pl.pallas_call(
        _k6_body,
        grid=(N // BR,),
        in_specs=[
            pl.BlockSpec((BR, D), lambda i: (i, 0)),
            pl.BlockSpec((N, D), lambda i: (0, 0)),
        ],
        out_specs=pl.BlockSpec((1, 1), lambda i: (0, 0)),
        out_shape=jax.ShapeDtypeStruct((1, 1), jnp.float32),
    )(items_n, items_n)


def _pad2(x, r, c):
    return jnp.pad(x, ((0, r - x.shape[0]), (0, c - x.shape[1])))


def kernel(embedding, pos_embedding, w_1, w_2, glu1_W, glu1_b, glu2_W,
           att_W, att_b, w_item0, w_item1, adj_vals, adj_row, adj_col,
           session_item, session_len, reversed_sess_item, mask, tar):
    f32 = jnp.float32
    emb = _pad2(embedding, N, D)
    pos = _pad2(pos_embedding, 200, D)
    w1a = _pad2(w_1[:100, :], D, D)
    w1b = _pad2(w_1[100:, :], D, D)
    w2 = _pad2(w_2, D, 8)
    g1w = _pad2(glu1_W, D, D)
    g1b = _pad2(glu1_b[None, :], 1, D)
    g2w = _pad2(glu2_W, D, D)
    aw = _pad2(att_W, 8, D)
    ab = att_b.reshape(1, 1)
    w0 = _pad2(w_item0, D, D)
    w1 = _pad2(w_item1, D, D)

    hist_cols = jnp.concatenate(
        [adj_col, jnp.full((EP - E0,), DUMP_BIN, jnp.int32)]
    ).reshape(NT, NCH, 128)
    z8 = jnp.zeros((8 * NB,), f32)
    hist = _sc_hist(hist_cols, z8).reshape(NT * 8, NB)

    ecol = jnp.concatenate([adj_col, jnp.zeros((EP - E0,), jnp.int32)])
    erow = jnp.concatenate(
        [adj_row, jnp.full((EP - E0,), DUMP_ROW, jnp.int32)])
    packed = (erow * 16384 + ecol).reshape(NT, NCH, 128)
    z632 = jnp.zeros((632, D), f32)

    g0, inv_deg = _tc_prep0(hist, emb, w0, aw, ab)
    p0 = _sc_spmm(g0, packed, z632)
    hn1, g1 = _tc_prep1(p0, w1, aw, ab, inv_deg)
    p1 = _sc_spmm(g1, packed, z632)
    emb_i, items_n = _tc_final_items(p1, emb, hn1)

    rsi = reversed_sess_item.reshape(B * L)
    gidx = jnp.maximum(rsi - 1, 0)
    gidx2d = jnp.concatenate(
        [gidx, jnp.zeros((GP - B * L,), jnp.int32)]).reshape(NT, NCHG, 128)
    seq_h = _sc_gather(emb_i, gidx2d).reshape(GP, D)[:B * L]

    vmask = (rsi > 0).astype(f32).reshape(B * L, 1)
    maskf = mask.reshape(B * L, 1)
    sess = _tc_session(seq_h, vmask, maskf, session_len, pos, w1a, w1b,
                       g1w, g1b, g2w, w2)

    scores, loss_item = _tc_scores(sess, items_n, tar.reshape(B, 1))
    cl = _tc_cl(items_n)
    return (loss_item[0, 0], scores, cl[0, 0])


# speedup vs baseline: 1.1858x; 1.0016x over previous
import functools

import jax
import jax.numpy as jnp
from jax import lax
from jax.experimental import pallas as pl
from jax.experimental.pallas import tpu as pltpu
from jax.experimental.pallas import tpu_sc as plsc

N = 10000
D = 128
B = 512
L = 50
TOPK = 10
TEMP = 0.1
W_K = 10.0
CLW = 100.0

NB = 10112
DUMP_BIN = 10016
NR = 10112
DUMP_ROW = 10008
NT = 32
E0 = N * 32 + N
NCH = 82
EP = NT * NCH * 128
NCHG = 8
GP = NT * NCHG * 128

def _mesh():
    return plsc.VectorSubcoreMesh(core_axis_name="c", subcore_axis_name="s")


def _sc_hist(cols3d, z8):
    @functools.partial(
        pl.kernel,
        out_type=jax.ShapeDtypeStruct((NT, 8 * NB), jnp.float32),
        mesh=_mesh(),
        compiler_params=pltpu.CompilerParams(needs_layout_passes=False),
        scratch_types=[
            pltpu.VMEM((NCH, 128), jnp.int32),
            pltpu.VMEM((8 * NB,), jnp.float32),
        ],
    )
    def k(cols_hbm, z_hbm, out_hbm, iall_v, hist_v):
        wid = lax.axis_index("s") * 2 + lax.axis_index("c")
        pltpu.sync_copy(cols_hbm.at[wid], iall_v)
        pltpu.sync_copy(z_hbm, hist_v)
        lane = lax.iota(jnp.int32, 16)
        mask_lo = lane < 8
        mask_hi = jnp.logical_not(mask_lo)
        off_lo = lane * NB
        off_hi = (lane - 8) * NB
        ones = jnp.ones((16,), jnp.float32)

        def chunk(ch, carry):
            for j in range(8):
                idx16 = iall_v[ch, pl.ds(j * 16, 16)]
                plsc.addupdate_scatter(hist_v, [off_lo + idx16], ones,
                                       mask=mask_lo)
                plsc.addupdate_scatter(hist_v, [off_hi + idx16], ones,
                                       mask=mask_hi)
            return carry

        lax.fori_loop(0, NCH, chunk, 0)
        pltpu.sync_copy(hist_v, out_hbm.at[wid])

    return k(cols3d, z8)


def _sc_spmm(g, packed, z632):
    @functools.partial(
        pl.kernel,
        out_type=jax.ShapeDtypeStruct((2, NR, D), jnp.float32),
        mesh=_mesh(),
        compiler_params=pltpu.CompilerParams(needs_layout_passes=False),
        scratch_types=[
            pltpu.VMEM((NCH, 128), jnp.int32),
            pltpu.VMEM((128,), jnp.int32),
            pltpu.VMEM((128,), jnp.int32),
            pltpu.VMEM((128,), jnp.int32),
            pltpu.VMEM((128,), jnp.int32),
            pltpu.VMEM((128, D), jnp.float32),
            pltpu.VMEM((128, D), jnp.float32),
            pltpu.VMEM_SHARED((NR, D), jnp.float32),
            pltpu.SemaphoreType.DMA,
            pltpu.SemaphoreType.DMA,
            pltpu.SemaphoreType.DMA,
            pltpu.SemaphoreType.DMA,
        ],
    )
    def k(g_hbm, p_hbm, z_hbm, out_hbm, pall_v, colv0, colv1, rowv0, rowv1,
          buf0, buf1, acc, gs0, gs1, ss0, ss1):
        cid = lax.axis_index("c")
        sid = lax.axis_index("s")
        wid = sid * 2 + cid
        colv = (colv0, colv1)
        rowv = (rowv0, rowv1)
        buf = (buf0, buf1)
        gsem = (gs0, gs1)
        ssem = (ss0, ss1)
        pltpu.sync_copy(p_hbm.at[wid], pall_v)
        pltpu.sync_copy(z_hbm, acc.at[pl.ds(sid * 632, 632)])

        def unpack(j, b):
            for t in range(8):
                pk = pall_v[j, pl.ds(t * 16, 16)]
                colv[b][pl.ds(t * 16, 16)] = jnp.bitwise_and(pk, 16383)
                rowv[b][pl.ds(t * 16, 16)] = jnp.right_shift(pk, 14)

        plsc.subcore_barrier()
        unpack(0, 0)
        pltpu.async_copy(g_hbm.at[colv0], buf0, gs0)

        def pair(j0, carry):
            for b in range(2):
                nb = 1 - b
                j = j0 * 2 + b

                @pl.when(j + 1 < NCH)
                def _():
                    @pl.when(j >= 1)
                    def _():
                        pltpu.make_async_copy(buf[nb], acc.at[rowv[nb]],
                                              ssem[nb]).wait()

                    unpack(j + 1, nb)
                    pltpu.async_copy(g_hbm.at[colv[nb]], buf[nb], gsem[nb])

                pltpu.make_async_copy(g_hbm.at[colv[b]], buf[b],
                                      gsem[b]).wait()
                pltpu.async_copy(buf[b], acc.at[rowv[b]], ssem[b],
                                add=True)
            return carry

        lax.fori_loop(0, NCH // 2, pair, 0)
        pltpu.make_async_copy(buf0, acc.at[rowv0], ss0).wait()
        pltpu.make_async_copy(buf1, acc.at[rowv1], ss1).wait()
        plsc.subcore_barrier()
        pltpu.sync_copy(acc.at[pl.ds(sid * 632, 632)],
                        out_hbm.at[cid, pl.ds(sid * 632, 632)])

    return k(g, packed, z632)


def _sc_gather(tbl, gidx2d):
    @functools.partial(
        pl.kernel,
        out_type=jax.ShapeDtypeStruct((NT * NCHG, 128, D), jnp.float32),
        mesh=_mesh(),
        compiler_params=pltpu.CompilerParams(needs_layout_passes=False),
        scratch_types=[
            pltpu.VMEM((NCHG, 128), jnp.int32),
            pltpu.VMEM((128, D), jnp.float32),
            pltpu.VMEM((128, D), jnp.float32),
            pltpu.SemaphoreType.DMA,
            pltpu.SemaphoreType.DMA,
            pltpu.SemaphoreType.DMA,
            pltpu.SemaphoreType.DMA,
        ],
    )
    def k(tbl_hbm, gidx_hbm, out_hbm, iall_v, buf0, buf1,
          gs0, gs1, ws0, ws1):
        wid = lax.axis_index("s") * 2 + lax.axis_index("c")
        base0 = wid * NCHG
        pltpu.sync_copy(gidx_hbm.at[wid], iall_v)

        def chunk(j, carry):
            pltpu.async_copy(tbl_hbm.at[iall_v.at[j]], buf0, gs0).wait()
            pltpu.sync_copy(buf0, out_hbm.at[base0 + j])
            return carry

        lax.fori_loop(0, NCHG, chunk, 0)

    return k(tbl, gidx2d)


def _l2n(x):
    n = jnp.sqrt(jnp.sum(x * x, axis=-1, keepdims=True))
    return x / jnp.maximum(n, 1e-12)


def _attw(emb, att_w, att_b):
    s = lax.dot_general(emb, att_w, (((1,), (1,)), ((), ())))[:, 0:1] + att_b
    m = jnp.max(s, axis=0, keepdims=True)
    e = jnp.exp(s - m)
    return e / jnp.sum(e, axis=0, keepdims=True)


def _prep_g(emb, w, att_w, att_b, inv_deg):
    h = lax.dot_general(emb, w, (((1,), (1,)), ((), ())))
    return h * _attw(emb, att_w, att_b) * inv_deg


def _k1_body(hist_ref, emb_ref, w_ref, aw_ref, ab_ref, g_ref, invdeg_ref):
    ones = jnp.ones((NT * 8, 8), jnp.float32)
    deg = lax.dot_general(hist_ref[...], ones, (((0,), (0,)), ((), ())))
    inv_deg = 1.0 / deg[:N, 0:1]
    invdeg_ref[...] = inv_deg
    g_ref[...] = _prep_g(emb_ref[...], w_ref[...], aw_ref[...],
                         ab_ref[0, 0], inv_deg)


def _tc_prep0(hist, emb, w0, att_w, att_b):
    return pl.pallas_call(
        _k1_body,
        out_shape=[jax.ShapeDtypeStruct((N, D), jnp.float32),
                   jax.ShapeDtypeStruct((N, 1), jnp.float32)],
    )(hist, emb, w0, att_w, att_b)


def _k2_body(p_ref, w_ref, aw_ref, ab_ref, invdeg_ref, hn_ref, g_ref):
    h = p_ref[0, :N, :] + p_ref[1, :N, :]
    hn = _l2n(h)
    hn_ref[...] = hn
    g_ref[...] = _prep_g(hn, w_ref[...], aw_ref[...], ab_ref[0, 0],
                         invdeg_ref[...])


def _tc_prep1(p, w1, att_w, att_b, inv_deg):
    return pl.pallas_call(
        _k2_body,
        out_shape=[jax.ShapeDtypeStruct((N, D), jnp.float32),
                   jax.ShapeDtypeStruct((N, D), jnp.float32)],
    )(p, w1, att_w, att_b, inv_deg)


def _k3_body(p_ref, emb_ref, hn1_ref, embi_ref, itemsn_ref):
    h2 = p_ref[0, :N, :] + p_ref[1, :N, :]
    hn2 = _l2n(h2)
    emb_i = (emb_ref[...] + hn1_ref[...] + hn2) * (1.0 / 3.0)
    embi_ref[...] = emb_i
    itemsn_ref[...] = _l2n(emb_i)


def _tc_final_items(p, emb, hn1):
    return pl.pallas_call(
        _k3_body,
        out_shape=[jax.ShapeDtypeStruct((N, D), jnp.float32),
                   jax.ShapeDtypeStruct((N, D), jnp.float32)],
    )(p, emb, hn1)


SB = 128
SR = SB * L


def _k4_body(sh_ref, vm_ref, mk_ref, len_ref, pos_ref, w1a_ref, w1b_ref,
             g1w_ref, g1b_ref, g2w_ref, w2_ref, sess_ref):
    sh = sh_ref[...] * vm_ref[...]
    sh3 = jnp.reshape(sh, (SB, L, D))
    hs = jnp.sum(sh3, axis=1) / len_ref[...]
    pos_part = lax.dot_general(pos_ref[:L, :], w1a_ref[...],
                               (((1,), (0,)), ((), ())))
    nh_lin = lax.dot_general(sh, w1b_ref[...], (((1,), (0,)), ((), ())))
    nh_lin = nh_lin + jnp.reshape(
        jnp.broadcast_to(pos_part[None], (SB, L, D)), (SR, D))
    nh = jnp.tanh(nh_lin)
    hsg = lax.dot_general(hs, g2w_ref[...], (((1,), (1,)), ((), ())))
    hsg = jnp.reshape(jnp.broadcast_to(hsg[:, None, :], (SB, L, D)), (SR, D))
    pre = lax.dot_general(nh, g1w_ref[...], (((1,), (1,)), ((), ())))
    nh2 = jax.nn.sigmoid(pre + g1b_ref[...] + hsg)
    beta = lax.dot_general(nh2, w2_ref[...], (((1,), (0,)), ((), ())))[:, 0:1]
    beta = beta * mk_ref[...]
    sel = jnp.sum(jnp.reshape(beta * sh, (SB, L, D)), axis=1)
    sess_ref[...] = W_K * _l2n(sel)


def _tc_session(seq_h, vmask, maskf, slen, pos, w1a, w1b, g1w, g1b, g2w, w2):
    nblk = B // SB
    full = lambda s: pl.BlockSpec(s, lambda i: (0, 0))
    return pl.pallas_call(
        _k4_body,
        grid=(nblk,),
        in_specs=[
            pl.BlockSpec((SR, D), lambda i: (i, 0)),
            pl.BlockSpec((SR, 1), lambda i: (i, 0)),
            pl.BlockSpec((SR, 1), lambda i: (i, 0)),
            pl.BlockSpec((SB, 1), lambda i: (i, 0)),
            full((200, D)), full((D, D)), full((D, D)), full((D, D)),
            full((1, D)), full((D, D)), full((D, 8)),
        ],
        out_specs=pl.BlockSpec((SB, D), lambda i: (i, 0)),
        out_shape=jax.ShapeDtypeStruct((B, D), jnp.float32),
    )(seq_h, vmask, maskf, slen, pos, w1a, w1b, g1w, g1b, g2w, w2)


def _k5_body(sess_ref, items_ref, tar_ref, scores_ref, loss_ref):
    sc = lax.dot_general(sess_ref[...], items_ref[...],
                         (((1,), (1,)), ((), ())))
    scores_ref[...] = sc
    m = jnp.max(sc, axis=1, keepdims=True)
    lse = jnp.log(jnp.sum(jnp.exp(sc - m), axis=1, keepdims=True)) + m
    col = lax.broadcasted_iota(jnp.int32, (B, N), 1)
    pick = jnp.sum(jnp.where(col == tar_ref[...], sc, 0.0), axis=1,
                   keepdims=True)
    loss_ref[...] = jnp.reshape(jnp.mean(lse - pick), (1, 1))


def _tc_scores(sess, items_n, tar2d):
    return pl.pallas_call(
        _k5_body,
        out_shape=[jax.ShapeDtypeStruct((B, N), jnp.float32),
                   jax.ShapeDtypeStruct((1, 1), jnp.float32)],
    )(sess, items_n, tar2d)


BR = 400


def _k6_body(x_ref, items_ref, cl_ref):
    lg = lax.dot_general(x_ref[...], items_ref[...],
                         (((1,), (1,)), ((), ()))) * (1.0 / TEMP)
    m0 = jnp.max(lg, axis=1, keepdims=True)
    acc = jnp.ones((BR, 1), jnp.float32)
    cur = lg
    mprev = m0
    for _ in range(TOPK - 1):
        cur = jnp.where(cur == mprev, -1e30, cur)
        mk = jnp.max(cur, axis=1, keepdims=True)
        acc = acc + jnp.exp(mk - m0)
        mprev = mk
    part = jnp.reshape(jnp.sum(jnp.log(acc)) * (CLW / N), (1, 1))

    @pl.when(pl.program_id(0) == 0)
    def _():
        cl_ref[...] = jnp.zeros((1, 1), jnp.float32)

    cl_ref[...] += part


def _tc_cl(items_n):
    return pl.pallas_call(
        _k6_body,
        grid=(N // BR,),
        in_specs=[
            pl.BlockSpec((BR, D), lambda i: (i, 0)),
            pl.BlockSpec((N, D), lambda i: (0, 0)),
        ],
        out_specs=pl.BlockSpec((1, 1), lambda i: (0, 0)),
        out_shape=jax.ShapeDtypeStruct((1, 1), jnp.float32),
    )(items_n, items_n)


def _pad2(x, r, c):
    return jnp.pad(x, ((0, r - x.shape[0]), (0, c - x.shape[1])))


def kernel(embedding, pos_embedding, w_1, w_2, glu1_W, glu1_b, glu2_W,
           att_W, att_b, w_item0, w_item1, adj_vals, adj_row, adj_col,
           session_item, session_len, reversed_sess_item, mask, tar):
    f32 = jnp.float32
    emb = _pad2(embedding, N, D)
    pos = _pad2(pos_embedding, 200, D)
    w1a = _pad2(w_1[:100, :], D, D)
    w1b = _pad2(w_1[100:, :], D, D)
    w2 = _pad2(w_2, D, 8)
    g1w = _pad2(glu1_W, D, D)
    g1b = _pad2(glu1_b[None, :], 1, D)
    g2w = _pad2(glu2_W, D, D)
    aw = _pad2(att_W, 8, D)
    ab = att_b.reshape(1, 1)
    w0 = _pad2(w_item0, D, D)
    w1 = _pad2(w_item1, D, D)

    hist_cols = jnp.concatenate(
        [adj_col, jnp.full((EP - E0,), DUMP_BIN, jnp.int32)]
    ).reshape(NT, NCH, 128)
    z8 = jnp.zeros((8 * NB,), f32)
    hist = _sc_hist(hist_cols, z8).reshape(NT * 8, NB)

    ecol = jnp.concatenate([adj_col, jnp.zeros((EP - E0,), jnp.int32)])
    erow = jnp.concatenate(
        [adj_row, jnp.full((EP - E0,), DUMP_ROW, jnp.int32)])
    packed = (erow * 16384 + ecol).reshape(NT, NCH, 128)
    z632 = jnp.zeros((632, D), f32)

    g0, inv_deg = _tc_prep0(hist, emb, w0, aw, ab)
    p0 = _sc_spmm(g0, packed, z632)
    hn1, g1 = _tc_prep1(p0, w1, aw, ab, inv_deg)
    p1 = _sc_spmm(g1, packed, z632)
    emb_i, items_n = _tc_final_items(p1, emb, hn1)

    rsi = reversed_sess_item.reshape(B * L)
    gidx = jnp.maximum(rsi - 1, 0)
    gidx2d = jnp.concatenate(
        [gidx, jnp.zeros((GP - B * L,), jnp.int32)]).reshape(NT, NCHG, 128)
    seq_h = _sc_gather(emb_i, gidx2d).reshape(GP, D)[:B * L]

    vmask = (rsi > 0).astype(f32).reshape(B * L, 1)
    maskf = mask.reshape(B * L, 1)
    sess = _tc_session(seq_h, vmask, maskf, session_len, pos, w1a, w1b,
                       g1w, g1b, g2w, w2)

    scores, loss_item = _tc_scores(sess, items_n, tar.reshape(B, 1))
    cl = _tc_cl(items_n)
    return (loss_item[0, 0], scores, cl[0, 0])


# speedup vs baseline: 1.1873x; 1.0013x over previous
import functools

import jax
import jax.numpy as jnp
from jax import lax
from jax.experimental import pallas as pl
from jax.experimental.pallas import tpu as pltpu
from jax.experimental.pallas import tpu_sc as plsc

N = 10000
D = 128
B = 512
L = 50
TOPK = 10
TEMP = 0.1
W_K = 10.0
CLW = 100.0

NB = 10112
DUMP_BIN = 10016
NR = 10112
DUMP_ROW = 10008
NT = 32
E0 = N * 32 + N
NCH = 82
EP = NT * NCH * 128
NCHG = 8
GP = NT * NCHG * 128

def _mesh():
    return plsc.VectorSubcoreMesh(core_axis_name="c", subcore_axis_name="s")


def _sc_hist(cols3d, z8):
    @functools.partial(
        pl.kernel,
        out_type=jax.ShapeDtypeStruct((NT, 8 * NB), jnp.float32),
        mesh=_mesh(),
        compiler_params=pltpu.CompilerParams(needs_layout_passes=False),
        scratch_types=[
            pltpu.VMEM((NCH, 128), jnp.int32),
            pltpu.VMEM((8 * NB,), jnp.float32),
        ],
    )
    def k(cols_hbm, z_hbm, out_hbm, iall_v, hist_v):
        wid = lax.axis_index("s") * 2 + lax.axis_index("c")
        pltpu.sync_copy(cols_hbm.at[wid], iall_v)
        pltpu.sync_copy(z_hbm, hist_v)
        lane = lax.iota(jnp.int32, 16)
        mask_lo = lane < 8
        mask_hi = jnp.logical_not(mask_lo)
        off_lo = lane * NB
        off_hi = (lane - 8) * NB
        ones = jnp.ones((16,), jnp.float32)

        def chunk(ch, carry):
            for j in range(8):
                idx16 = iall_v[ch, pl.ds(j * 16, 16)]
                plsc.addupdate_scatter(hist_v, [off_lo + idx16], ones,
                                       mask=mask_lo)
                plsc.addupdate_scatter(hist_v, [off_hi + idx16], ones,
                                       mask=mask_hi)
            return carry

        lax.fori_loop(0, NCH, chunk, 0)
        pltpu.sync_copy(hist_v, out_hbm.at[wid])

    return k(cols3d, z8)


def _sc_spmm(g, packed, z632):
    @functools.partial(
        pl.kernel,
        out_type=jax.ShapeDtypeStruct((2, NR, D), jnp.float32),
        mesh=_mesh(),
        compiler_params=pltpu.CompilerParams(needs_layout_passes=False),
        scratch_types=[
            pltpu.VMEM((NCH, 128), jnp.int32),
            pltpu.VMEM((128,), jnp.int32),
            pltpu.VMEM((128,), jnp.int32),
            pltpu.VMEM((128,), jnp.int32),
            pltpu.VMEM((128,), jnp.int32),
            pltpu.VMEM((128, D), jnp.float32),
            pltpu.VMEM((128, D), jnp.float32),
            pltpu.VMEM_SHARED((NR, D), jnp.float32),
            pltpu.SemaphoreType.DMA,
            pltpu.SemaphoreType.DMA,
            pltpu.SemaphoreType.DMA,
            pltpu.SemaphoreType.DMA,
        ],
    )
    def k(g_hbm, p_hbm, z_hbm, out_hbm, pall_v, colv0, colv1, rowv0, rowv1,
          buf0, buf1, acc, gs0, gs1, ss0, ss1):
        cid = lax.axis_index("c")
        sid = lax.axis_index("s")
        wid = sid * 2 + cid
        colv = (colv0, colv1)
        rowv = (rowv0, rowv1)
        buf = (buf0, buf1)
        gsem = (gs0, gs1)
        ssem = (ss0, ss1)
        pltpu.sync_copy(p_hbm.at[wid], pall_v)
        pltpu.sync_copy(z_hbm, acc.at[pl.ds(sid * 632, 632)])

        def unpack(j, b):
            for t in range(8):
                pk = pall_v[j, pl.ds(t * 16, 16)]
                colv[b][pl.ds(t * 16, 16)] = jnp.bitwise_and(pk, 16383)
                rowv[b][pl.ds(t * 16, 16)] = jnp.right_shift(pk, 14)

        plsc.subcore_barrier()
        unpack(0, 0)
        pltpu.async_copy(g_hbm.at[colv0], buf0, gs0)

        def pair(j0, carry):
            for b in range(2):
                nb = 1 - b
                j = j0 * 2 + b

                @pl.when(j + 1 < NCH)
                def _():
                    @pl.when(j >= 1)
                    def _():
                        pltpu.make_async_copy(buf[nb], acc.at[rowv[nb]],
                                              ssem[nb]).wait()

                    unpack(j + 1, nb)
                    pltpu.async_copy(g_hbm.at[colv[nb]], buf[nb], gsem[nb])

                pltpu.make_async_copy(g_hbm.at[colv[b]], buf[b],
                                      gsem[b]).wait()
                pltpu.async_copy(buf[b], acc.at[rowv[b]], ssem[b],
                                add=True)
            return carry

        lax.fori_loop(0, NCH // 2, pair, 0)
        pltpu.make_async_copy(buf0, acc.at[rowv0], ss0).wait()
        pltpu.make_async_copy(buf1, acc.at[rowv1], ss1).wait()
        plsc.subcore_barrier()
        pltpu.sync_copy(acc.at[pl.ds(sid * 632, 632)],
                        out_hbm.at[cid, pl.ds(sid * 632, 632)])

    return k(g, packed, z632)


def _sc_gather(tbl, gidx2d):
    @functools.partial(
        pl.kernel,
        out_type=jax.ShapeDtypeStruct((NT * NCHG, 128, D), jnp.float32),
        mesh=_mesh(),
        compiler_params=pltpu.CompilerParams(needs_layout_passes=False),
        scratch_types=[
            pltpu.VMEM((NCHG, 128), jnp.int32),
            pltpu.VMEM((128, D), jnp.float32),
            pltpu.VMEM((128, D), jnp.float32),
            pltpu.SemaphoreType.DMA,
            pltpu.SemaphoreType.DMA,
            pltpu.SemaphoreType.DMA,
            pltpu.SemaphoreType.DMA,
        ],
    )
    def k(tbl_hbm, gidx_hbm, out_hbm, iall_v, buf0, buf1,
          gs0, gs1, ws0, ws1):
        wid = lax.axis_index("s") * 2 + lax.axis_index("c")
        base0 = wid * NCHG
        pltpu.sync_copy(gidx_hbm.at[wid], iall_v)

        def chunk(j, carry):
            pltpu.async_copy(tbl_hbm.at[iall_v.at[j]], buf0, gs0).wait()
            pltpu.sync_copy(buf0, out_hbm.at[base0 + j])
            return carry

        lax.fori_loop(0, NCHG, chunk, 0)

    return k(tbl, gidx2d)


def _l2n(x):
    n = jnp.sqrt(jnp.sum(x * x, axis=-1, keepdims=True))
    return x / jnp.maximum(n, 1e-12)


def _attw(emb, att_w, att_b):
    s = lax.dot_general(emb, att_w, (((1,), (1,)), ((), ())))[:, 0:1] + att_b
    m = jnp.max(s, axis=0, keepdims=True)
    e = jnp.exp(s - m)
    return e / jnp.sum(e, axis=0, keepdims=True)


def _prep_g(emb, w, att_w, att_b, inv_deg):
    h = lax.dot_general(emb, w, (((1,), (1,)), ((), ())))
    return h * _attw(emb, att_w, att_b) * inv_deg


def _k1_body(hist_ref, emb_ref, w_ref, aw_ref, ab_ref, g_ref, invdeg_ref):
    ones = jnp.ones((NT * 8, 8), jnp.float32)
    deg = lax.dot_general(hist_ref[...], ones, (((0,), (0,)), ((), ())))
    inv_deg = 1.0 / deg[:N, 0:1]
    invdeg_ref[...] = inv_deg
    g_ref[...] = _prep_g(emb_ref[...], w_ref[...], aw_ref[...],
                         ab_ref[0, 0], inv_deg)


def _tc_prep0(hist, emb, w0, att_w, att_b):
    return pl.pallas_call(
        _k1_body,
        out_shape=[jax.ShapeDtypeStruct((N, D), jnp.float32),
                   jax.ShapeDtypeStruct((N, 1), jnp.float32)],
    )(hist, emb, w0, att_w, att_b)


def _k2_body(p_ref, w_ref, aw_ref, ab_ref, invdeg_ref, hn_ref, g_ref):
    h = p_ref[0, :N, :] + p_ref[1, :N, :]
    hn = _l2n(h)
    hn_ref[...] = hn
    g_ref[...] = _prep_g(hn, w_ref[...], aw_ref[...], ab_ref[0, 0],
                         invdeg_ref[...])


def _tc_prep1(p, w1, att_w, att_b, inv_deg):
    return pl.pallas_call(
        _k2_body,
        out_shape=[jax.ShapeDtypeStruct((N, D), jnp.float32),
                   jax.ShapeDtypeStruct((N, D), jnp.float32)],
    )(p, w1, att_w, att_b, inv_deg)


def _k3_body(p_ref, emb_ref, hn1_ref, embi_ref, itemsn_ref):
    h2 = p_ref[0, :N, :] + p_ref[1, :N, :]
    hn2 = _l2n(h2)
    emb_i = (emb_ref[...] + hn1_ref[...] + hn2) * (1.0 / 3.0)
    embi_ref[...] = emb_i
    itemsn_ref[...] = _l2n(emb_i)


def _tc_final_items(p, emb, hn1):
    return pl.pallas_call(
        _k3_body,
        out_shape=[jax.ShapeDtypeStruct((N, D), jnp.float32),
                   jax.ShapeDtypeStruct((N, D), jnp.float32)],
    )(p, emb, hn1)


SB = 128
SR = SB * L


def _k4_body(sh_ref, vm_ref, mk_ref, len_ref, pos_ref, w1a_ref, w1b_ref,
             g1w_ref, g1b_ref, g2w_ref, w2_ref, sess_ref):
    sh = sh_ref[...] * vm_ref[...]
    sh3 = jnp.reshape(sh, (SB, L, D))
    hs = jnp.sum(sh3, axis=1) / len_ref[...]
    pos_part = lax.dot_general(pos_ref[:L, :], w1a_ref[...],
                               (((1,), (0,)), ((), ())))
    nh_lin = lax.dot_general(sh, w1b_ref[...], (((1,), (0,)), ((), ())))
    nh_lin = nh_lin + jnp.reshape(
        jnp.broadcast_to(pos_part[None], (SB, L, D)), (SR, D))
    nh = jnp.tanh(nh_lin)
    hsg = lax.dot_general(hs, g2w_ref[...], (((1,), (1,)), ((), ())))
    hsg = jnp.reshape(jnp.broadcast_to(hsg[:, None, :], (SB, L, D)), (SR, D))
    pre = lax.dot_general(nh, g1w_ref[...], (((1,), (1,)), ((), ())))
    nh2 = jax.nn.sigmoid(pre + g1b_ref[...] + hsg)
    beta = lax.dot_general(nh2, w2_ref[...], (((1,), (0,)), ((), ())))[:, 0:1]
    beta = beta * mk_ref[...]
    sel = jnp.sum(jnp.reshape(beta * sh, (SB, L, D)), axis=1)
    sess_ref[...] = W_K * _l2n(sel)


def _tc_session(seq_h, vmask, maskf, slen, pos, w1a, w1b, g1w, g1b, g2w, w2):
    nblk = B // SB
    full = lambda s: pl.BlockSpec(s, lambda i: (0, 0))
    return pl.pallas_call(
        _k4_body,
        grid=(nblk,),
        in_specs=[
            pl.BlockSpec((SR, D), lambda i: (i, 0)),
            pl.BlockSpec((SR, 1), lambda i: (i, 0)),
            pl.BlockSpec((SR, 1), lambda i: (i, 0)),
            pl.BlockSpec((SB, 1), lambda i: (i, 0)),
            full((200, D)), full((D, D)), full((D, D)), full((D, D)),
            full((1, D)), full((D, D)), full((D, 8)),
        ],
        out_specs=pl.BlockSpec((SB, D), lambda i: (i, 0)),
        out_shape=jax.ShapeDtypeStruct((B, D), jnp.float32),
    )(seq_h, vmask, maskf, slen, pos, w1a, w1b, g1w, g1b, g2w, w2)


def _k5_body(sess_ref, items_ref, tar_ref, scores_ref, loss_ref):
    sc = lax.dot_general(sess_ref[...], items_ref[...],
                         (((1,), (1,)), ((), ())))
    scores_ref[...] = sc
    m = jnp.max(sc, axis=1, keepdims=True)
    lse = jnp.log(jnp.sum(jnp.exp(sc - m), axis=1, keepdims=True)) + m
    col = lax.broadcasted_iota(jnp.int32, (B, N), 1)
    pick = jnp.sum(jnp.where(col == tar_ref[...], sc, 0.0), axis=1,
                   keepdims=True)
    loss_ref[...] = jnp.reshape(jnp.mean(lse - pick), (1, 1))


def _tc_scores(sess, items_n, tar2d):
    return pl.pallas_call(
        _k5_body,
        out_shape=[jax.ShapeDtypeStruct((B, N), jnp.float32),
                   jax.ShapeDtypeStruct((1, 1), jnp.float32)],
    )(sess, items_n, tar2d)


BR = 400


def _k6_body(x_ref, items_ref, cl_ref):
    lg = lax.dot_general(x_ref[...], items_ref[...],
                         (((1,), (1,)), ((), ()))) * (1.0 / TEMP)
    m0 = jnp.max(lg, axis=1, keepdims=True)
    acc = jnp.ones((BR, 1), jnp.float32)
    mprev = m0
    for _ in range(TOPK - 1):
        mk = jnp.max(jnp.where(lg < mprev, lg, -1e30), axis=1, keepdims=True)
        acc = acc + jnp.exp(mk - m0)
        mprev = mk
    part = jnp.reshape(jnp.sum(jnp.log(acc)) * (CLW / N), (1, 1))

    @pl.when(pl.program_id(0) == 0)
    def _():
        cl_ref[...] = jnp.zeros((1, 1), jnp.float32)

    cl_ref[...] += part


def _tc_cl(items_n):
    return pl.pallas_call(
        _k6_body,
        grid=(N // BR,),
        in_specs=[
            pl.BlockSpec((BR, D), lambda i: (i, 0)),
            pl.BlockSpec((N, D), lambda i: (0, 0)),
        ],
        out_specs=pl.BlockSpec((1, 1), lambda i: (0, 0)),
        out_shape=jax.ShapeDtypeStruct((1, 1), jnp.float32),
    )(items_n, items_n)


def _pad2(x, r, c):
    return jnp.pad(x, ((0, r - x.shape[0]), (0, c - x.shape[1])))


def kernel(embedding, pos_embedding, w_1, w_2, glu1_W, glu1_b, glu2_W,
           att_W, att_b, w_item0, w_item1, adj_vals, adj_row, adj_col,
           session_item, session_len, reversed_sess_item, mask, tar):
    f32 = jnp.float32
    emb = _pad2(embedding, N, D)
    pos = _pad2(pos_embedding, 200, D)
    w1a = _pad2(w_1[:100, :], D, D)
    w1b = _pad2(w_1[100:, :], D, D)
    w2 = _pad2(w_2, D, 8)
    g1w = _pad2(glu1_W, D, D)
    g1b = _pad2(glu1_b[None, :], 1, D)
    g2w = _pad2(glu2_W, D, D)
    aw = _pad2(att_W, 8, D)
    ab = att_b.reshape(1, 1)
    w0 = _pad2(w_item0, D, D)
    w1 = _pad2(w_item1, D, D)

    hist_cols = jnp.concatenate(
        [adj_col, jnp.full((EP - E0,), DUMP_BIN, jnp.int32)]
    ).reshape(NT, NCH, 128)
    z8 = jnp.zeros((8 * NB,), f32)
    hist = _sc_hist(hist_cols, z8).reshape(NT * 8, NB)

    ecol = jnp.concatenate([adj_col, jnp.zeros((EP - E0,), jnp.int32)])
    erow = jnp.concatenate(
        [adj_row, jnp.full((EP - E0,), DUMP_ROW, jnp.int32)])
    packed = (erow * 16384 + ecol).reshape(NT, NCH, 128)
    z632 = jnp.zeros((632, D), f32)

    g0, inv_deg = _tc_prep0(hist, emb, w0, aw, ab)
    p0 = _sc_spmm(g0, packed, z632)
    hn1, g1 = _tc_prep1(p0, w1, aw, ab, inv_deg)
    p1 = _sc_spmm(g1, packed, z632)
    emb_i, items_n = _tc_final_items(p1, emb, hn1)

    rsi = reversed_sess_item.reshape(B * L)
    gidx = jnp.maximum(rsi - 1, 0)
    gidx2d = jnp.concatenate(
        [gidx, jnp.zeros((GP - B * L,), jnp.int32)]).reshape(NT, NCHG, 128)
    seq_h = _sc_gather(emb_i, gidx2d).reshape(GP, D)[:B * L]

    vmask = (rsi > 0).astype(f32).reshape(B * L, 1)
    maskf = mask.reshape(B * L, 1)
    sess = _tc_session(seq_h, vmask, maskf, session_len, pos, w1a, w1b,
                       g1w, g1b, g2w, w2)

    scores, loss_item = _tc_scores(sess, items_n, tar.reshape(B, 1))
    cl = _tc_cl(items_n)
    return (loss_item[0, 0], scores, cl[0, 0])
